# baseline (speedup 1.0000x reference)
import numpy as np
import jax
import jax.numpy as jnp
from jax.experimental import pallas as pl
from jax.experimental.pallas import tpu as pltpu

N = 8192
N1 = 64
N2 = 128
NSEQ = 64 * 32
TOPK = 5
B = 16

_HI = jax.lax.Precision.HIGHEST


def _make_tables():
    k1 = np.arange(N1)
    n1 = np.arange(N1)
    C1 = np.cos(2 * np.pi * np.outer(k1, n1) / N1).astype(np.float32)
    S1 = np.sin(2 * np.pi * np.outer(k1, n1) / N1).astype(np.float32)
    n2 = np.arange(N2)
    Ct = np.cos(2 * np.pi * np.outer(k1, n2) / N).astype(np.float32)
    St = np.sin(2 * np.pi * np.outer(k1, n2) / N).astype(np.float32)
    k2 = np.arange(N2)
    C2 = np.cos(2 * np.pi * np.outer(n2, k2) / N2).astype(np.float32)
    S2 = np.sin(2 * np.pi * np.outer(n2, k2) / N2).astype(np.float32)
    fgrid = k1[:, None] + N1 * k2[None, :]
    valid = ((fgrid >= 1) & (fgrid <= N // 2)).astype(np.float32)
    return C1, S1, Ct, St, C2, S2, valid


_TABLES = _make_tables()

_DN_T = (((1,), (1,)), ((), ()))


def _dft_decomp_kernel(x_ref, c1_ref, s1_ref, ct_ref, st_ref, c2_ref,
                       s2_ref, valid_ref, season_ref, trend_ref):
    X = x_ref[...]
    C1 = c1_ref[...]
    S1 = s1_ref[...]
    Ct = ct_ref[...]
    St = st_ref[...]
    C2 = c2_ref[...]
    S2 = s2_ref[...]
    valid = valid_ref[...]

    yre_l = []
    yim_l = []
    for b in range(B):
        xb = X[b]
        yre_l.append(jax.lax.dot(C1, xb, precision=_HI)[None])
        yim_l.append(-jax.lax.dot(S1, xb, precision=_HI)[None])
    Yre = jnp.concatenate(yre_l, axis=0)
    Yim = jnp.concatenate(yim_l, axis=0)

    Ypre = Yre * Ct[None] + Yim * St[None]
    Ypim = Yim * Ct[None] - Yre * St[None]

    Ypre2 = Ypre.reshape(B * N1, N2)
    Ypim2 = Ypim.reshape(B * N1, N2)
    Zre2 = (jax.lax.dot(Ypre2, C2, precision=_HI)
            + jax.lax.dot(Ypim2, S2, precision=_HI))
    Zim2 = (jax.lax.dot(Ypim2, C2, precision=_HI)
            - jax.lax.dot(Ypre2, S2, precision=_HI))
    Zre = Zre2.reshape(B, N1, N2)
    Zim = Zim2.reshape(B, N1, N2)
    ZZ2 = jnp.concatenate([Zre2, Zim2], axis=0)

    mag = jnp.where(valid[None] > 0, Zre * Zre + Zim * Zim, -1.0)
    mag2 = mag.reshape(B * N1, N2)

    colmax = jnp.max(mag, axis=1)

    iota_b = jax.lax.broadcasted_iota(jnp.int32, (B, 1, B), 2)
    eye_b = (iota_b == jax.lax.broadcasted_iota(jnp.int32, (B, 1, B), 0)
             ).astype(jnp.float32)
    iota64 = jax.lax.broadcasted_iota(jnp.int32, (B, N1), 1).astype(
        jnp.float32)
    iota128 = jax.lax.broadcasted_iota(jnp.int32, (B, N2), 1).astype(
        jnp.float32)

    season = jnp.zeros((B, N1, N2), jnp.float32)
    taken = []
    for _ in range(TOPK):
        m = jnp.max(colmax, axis=1, keepdims=True)
        selc = (colmax == m).astype(jnp.float32)
        k2f = jnp.sum(selc * iota128, axis=1, keepdims=True)

        emagd = jax.lax.dot_general(mag2, selc, _DN_T, precision=_HI)
        emag = jnp.sum(emagd.reshape(B, N1, B) * eye_b, axis=2)
        for (pk1, pk2) in taken:
            emag = jnp.where((pk2 == k2f) & (iota64 == pk1), -3.0, emag)

        rowmax = jnp.max(emag, axis=1, keepdims=True)
        rsel = (emag == rowmax).astype(jnp.float32)
        k1f = jnp.sum(rsel * iota64, axis=1, keepdims=True)
        taken.append((k1f, k2f))

        runner = jnp.max(jnp.where(rsel > 0, -3.0, emag), axis=1,
                         keepdims=True)
        colmax = jnp.where(selc > 0, runner, colmax)

        ed = jax.lax.dot_general(ZZ2, selc, _DN_T, precision=_HI)
        ecols = jnp.sum(ed.reshape(2, B, N1, B) * eye_b[None], axis=3)
        a2 = jnp.sum(rsel * ecols[0], axis=1, keepdims=True)
        b2 = jnp.sum(rsel * ecols[1], axis=1, keepdims=True)

        eps = jnp.where((k1f == 0.0) & (k2f == 64.0), 1.0, 2.0)

        ure = jax.lax.dot(rsel, C1, precision=_HI)
        uim = jax.lax.dot(rsel, S1, precision=_HI)
        twc = jax.lax.dot(rsel, Ct, precision=_HI)
        tws = jax.lax.dot(rsel, St, precision=_HI)
        c2v = jax.lax.dot(selc, C2, precision=_HI)
        s2v = jax.lax.dot(selc, S2, precision=_HI)
        wre = twc * c2v - tws * s2v
        wim = twc * s2v + tws * c2v
        scale = eps * (1.0 / N)
        cure = scale * (a2 * ure - b2 * uim)
        cuim = scale * (a2 * uim + b2 * ure)
        season = (season + cure[:, :, None] * wre[:, None, :]
                  - cuim[:, :, None] * wim[:, None, :])

    season_ref[...] = season
    trend_ref[...] = X - season


def _run(x3, interpret=False):
    nseq = x3.shape[0]
    grid = (nseq // B,)
    tabs = [jnp.asarray(t) for t in _TABLES]
    tab_specs = [pl.BlockSpec(t.shape, lambda i: (0,) * t.ndim)
                 for t in tabs]
    season3, trend3 = pl.pallas_call(
        _dft_decomp_kernel,
        grid=grid,
        in_specs=[pl.BlockSpec((B, N1, N2), lambda i: (i, 0, 0))] + tab_specs,
        out_specs=[pl.BlockSpec((B, N1, N2), lambda i: (i, 0, 0)),
                   pl.BlockSpec((B, N1, N2), lambda i: (i, 0, 0))],
        out_shape=[jax.ShapeDtypeStruct((nseq, N1, N2), jnp.float32),
                   jax.ShapeDtypeStruct((nseq, N1, N2), jnp.float32)],
        interpret=interpret,
    )(x3, *tabs)
    return season3, trend3


def kernel(x):
    bsz, ch, n = x.shape
    x3 = x.reshape(bsz * ch, N1, N2)
    season3, trend3 = _run(x3)
    return (season3.reshape(bsz, ch, n), trend3.reshape(bsz, ch, n))


# speedup vs baseline: 1.6329x; 1.6329x over previous
import numpy as np
import jax
import jax.numpy as jnp
from jax.experimental import pallas as pl
from jax.experimental.pallas import tpu as pltpu

N = 8192
N1 = 64
N2 = 128
NSEQ = 64 * 32
TOPK = 5
B = 32

_HI = jax.lax.Precision.HIGHEST


def _make_tables():
    k1 = np.arange(N1)
    n1 = np.arange(N1)
    C1 = np.cos(2 * np.pi * np.outer(k1, n1) / N1).astype(np.float32)
    S1 = np.sin(2 * np.pi * np.outer(k1, n1) / N1).astype(np.float32)
    n2 = np.arange(N2)
    Ct = np.cos(2 * np.pi * np.outer(k1, n2) / N).astype(np.float32)
    St = np.sin(2 * np.pi * np.outer(k1, n2) / N).astype(np.float32)
    k2 = np.arange(N2)
    C2 = np.cos(2 * np.pi * np.outer(n2, k2) / N2).astype(np.float32)
    S2 = np.sin(2 * np.pi * np.outer(n2, k2) / N2).astype(np.float32)
    fgrid = (k1[:, None] + N1 * k2[None, :]).astype(np.float32)
    valid = ((fgrid >= 1) & (fgrid <= N // 2)).astype(np.float32)
    return C1, S1, Ct, St, C2, S2, fgrid, valid


_TABLES = _make_tables()


def _dft_decomp_kernel(x_ref, c1_ref, s1_ref, ct_ref, st_ref, c2_ref,
                       s2_ref, fg_ref, valid_ref, season_ref, trend_ref):
    X = x_ref[...]
    C1 = c1_ref[...]
    S1 = s1_ref[...]
    Ct = ct_ref[...]
    St = st_ref[...]
    C2 = c2_ref[...]
    S2 = s2_ref[...]
    fg = fg_ref[...]
    valid = valid_ref[...]

    yre_l = []
    yim_l = []
    for b in range(B):
        xb = X[b]
        yre_l.append(jax.lax.dot(C1, xb, precision=_HI)[None])
        yim_l.append(-jax.lax.dot(S1, xb, precision=_HI)[None])
    Yre = jnp.concatenate(yre_l, axis=0)
    Yim = jnp.concatenate(yim_l, axis=0)

    Ypre = Yre * Ct[None] + Yim * St[None]
    Ypim = Yim * Ct[None] - Yre * St[None]

    Ypre2 = Ypre.reshape(B * N1, N2)
    Ypim2 = Ypim.reshape(B * N1, N2)
    Zre2 = (jax.lax.dot(Ypre2, C2, precision=_HI)
            + jax.lax.dot(Ypim2, S2, precision=_HI))
    Zim2 = (jax.lax.dot(Ypim2, C2, precision=_HI)
            - jax.lax.dot(Ypre2, S2, precision=_HI))
    Zre = Zre2.reshape(B, N1, N2)
    Zim = Zim2.reshape(B, N1, N2)

    mag = jnp.where(valid[None] > 0, Zre * Zre + Zim * Zim, -1.0)

    season = jnp.zeros((B, N1, N2), jnp.float32)
    for _ in range(TOPK):
        m = jnp.max(jnp.max(mag, axis=2, keepdims=True), axis=1,
                    keepdims=True)
        sel = (mag == m).astype(jnp.float32)
        a = jnp.sum(jnp.sum(sel * Zre, axis=2, keepdims=True), axis=1,
                    keepdims=True)
        bb = jnp.sum(jnp.sum(sel * Zim, axis=2, keepdims=True), axis=1,
                     keepdims=True)
        fsel = jnp.sum(jnp.sum(sel * fg[None], axis=2, keepdims=True),
                       axis=1, keepdims=True)
        k2f = jnp.floor(fsel * (1.0 / N1))
        k1f = fsel - N1 * k2f
        eps = jnp.where(fsel == float(N // 2), 1.0, 2.0)

        k1i = k1f.reshape(B, 1).astype(jnp.int32)
        k2i = k2f.reshape(B, 1).astype(jnp.int32)
        roh = (jax.lax.broadcasted_iota(jnp.int32, (B, N1), 1)
               == k1i).astype(jnp.float32)
        coh = (jax.lax.broadcasted_iota(jnp.int32, (B, N2), 1)
               == k2i).astype(jnp.float32)
        ure = jax.lax.dot(roh, C1, precision=_HI)
        uim = jax.lax.dot(roh, S1, precision=_HI)
        twc = jax.lax.dot(roh, Ct, precision=_HI)
        tws = jax.lax.dot(roh, St, precision=_HI)
        c2v = jax.lax.dot(coh, C2, precision=_HI)
        s2v = jax.lax.dot(coh, S2, precision=_HI)
        wre = twc * c2v - tws * s2v
        wim = twc * s2v + tws * c2v
        scale = (eps * (1.0 / N)).reshape(B, 1)
        a2 = a.reshape(B, 1)
        b2 = bb.reshape(B, 1)
        cure = scale * (a2 * ure - b2 * uim)
        cuim = scale * (a2 * uim + b2 * ure)
        season = (season + cure[:, :, None] * wre[:, None, :]
                  - cuim[:, :, None] * wim[:, None, :])
        mag = jnp.where(sel > 0, -1.0, mag)

    season_ref[...] = season
    trend_ref[...] = X - season


def _run(x3, interpret=False):
    nseq = x3.shape[0]
    grid = (nseq // B,)
    tabs = [jnp.asarray(t) for t in _TABLES]
    tab_specs = [pl.BlockSpec(t.shape, lambda i: (0,) * t.ndim)
                 for t in tabs]
    season3, trend3 = pl.pallas_call(
        _dft_decomp_kernel,
        grid=grid,
        in_specs=[pl.BlockSpec((B, N1, N2), lambda i: (i, 0, 0))] + tab_specs,
        out_specs=[pl.BlockSpec((B, N1, N2), lambda i: (i, 0, 0)),
                   pl.BlockSpec((B, N1, N2), lambda i: (i, 0, 0))],
        out_shape=[jax.ShapeDtypeStruct((nseq, N1, N2), jnp.float32),
                   jax.ShapeDtypeStruct((nseq, N1, N2), jnp.float32)],
        interpret=interpret,
    )(x3, *tabs)
    return season3, trend3


def kernel(x):
    bsz, ch, n = x.shape
    x3 = x.reshape(bsz * ch, N1, N2)
    season3, trend3 = _run(x3)
    return (season3.reshape(bsz, ch, n), trend3.reshape(bsz, ch, n))


# speedup vs baseline: 2.0490x; 1.2548x over previous
import numpy as np
import jax
import jax.numpy as jnp
from jax.experimental import pallas as pl
from jax.experimental.pallas import tpu as pltpu

N = 8192
N1 = 64
N2 = 128
NSEQ = 64 * 32
TOPK = 5
B = 32

_HI = jax.lax.Precision.HIGHEST


def _make_tables():
    k1 = np.arange(N1)
    n1 = np.arange(N1)
    C1 = np.cos(2 * np.pi * np.outer(k1, n1) / N1).astype(np.float32)
    S1 = np.sin(2 * np.pi * np.outer(k1, n1) / N1).astype(np.float32)
    n2 = np.arange(N2)
    Ct = np.cos(2 * np.pi * np.outer(k1, n2) / N).astype(np.float32)
    St = np.sin(2 * np.pi * np.outer(k1, n2) / N).astype(np.float32)
    k2 = np.arange(N2)
    C2 = np.cos(2 * np.pi * np.outer(n2, k2) / N2).astype(np.float32)
    S2 = np.sin(2 * np.pi * np.outer(n2, k2) / N2).astype(np.float32)
    fgrid = (k1[:, None] + N1 * k2[None, :]).astype(np.float32)
    valid = ((fgrid >= 1) & (fgrid <= N // 2)).astype(np.float32)
    BD1C = np.kron(np.eye(4, dtype=np.float32), C1)
    BD1S = np.kron(np.eye(4, dtype=np.float32), S1)
    C2S2 = np.concatenate([C2, S2], axis=1)
    C1S1 = np.concatenate([C1, S1], axis=1)
    CtSt = np.concatenate([Ct, St], axis=1)
    return C1, S1, Ct, St, C2, S2, fgrid, valid, BD1C, BD1S, C2S2, C1S1, CtSt


_TABLES = _make_tables()


def _dft_decomp_kernel(x_ref, c1_ref, s1_ref, ct_ref, st_ref, c2_ref,
                       s2_ref, fg_ref, valid_ref, bd1c_ref, bd1s_ref,
                       c2s2_ref, c1s1_ref, ctst_ref, season_ref,
                       trend_ref):
    X = x_ref[...]
    C1 = c1_ref[...]
    S1 = s1_ref[...]
    Ct = ct_ref[...]
    St = st_ref[...]
    C2 = c2_ref[...]
    S2 = s2_ref[...]
    fg = fg_ref[...]
    valid = valid_ref[...]
    BD1C = bd1c_ref[...]
    BD1S = bd1s_ref[...]
    C2S2 = c2s2_ref[...]
    C1S1 = c1s1_ref[...]
    CtSt = ctst_ref[...]

    yre_l = []
    yim_l = []
    for b in range(B):
        xb = X[b]
        yre_l.append(jax.lax.dot(C1, xb, precision=_HI)[None])
        yim_l.append(-jax.lax.dot(S1, xb, precision=_HI)[None])
    Yre = jnp.concatenate(yre_l, axis=0)
    Yim = jnp.concatenate(yim_l, axis=0)

    Ypre = Yre * Ct[None] + Yim * St[None]
    Ypim = Yim * Ct[None] - Yre * St[None]

    Ypre2 = Ypre.reshape(B * N1, N2)
    Ypim2 = Ypim.reshape(B * N1, N2)
    Pcs = jax.lax.dot(Ypre2, C2S2, precision=_HI)
    Qcs = jax.lax.dot(Ypim2, C2S2, precision=_HI)
    Zre2 = Pcs[:, :N2] + Qcs[:, N2:]
    Zim2 = Qcs[:, :N2] - Pcs[:, N2:]
    Zre = Zre2.reshape(B, N1, N2)
    Zim = Zim2.reshape(B, N1, N2)

    mag = jnp.where(valid[None] > 0, Zre * Zre + Zim * Zim, -1.0)

    season = jnp.zeros((B, N1, N2), jnp.float32)
    for _ in range(TOPK):
        m = jnp.max(mag, axis=(1, 2), keepdims=True)
        sel = (mag == m).astype(jnp.float32)
        a = jnp.sum(sel * Zre, axis=(1, 2), keepdims=True)
        bb = jnp.sum(sel * Zim, axis=(1, 2), keepdims=True)
        fsel = jnp.sum(sel * fg[None], axis=(1, 2), keepdims=True)
        k2f = jnp.floor(fsel * (1.0 / N1))
        k1f = fsel - N1 * k2f
        eps = jnp.where(fsel == float(N // 2), 1.0, 2.0)

        k1i = k1f.reshape(B, 1).astype(jnp.int32)
        k2i = k2f.reshape(B, 1).astype(jnp.int32)
        roh = (jax.lax.broadcasted_iota(jnp.int32, (B, N1), 1)
               == k1i).astype(jnp.float32)
        coh = (jax.lax.broadcasted_iota(jnp.int32, (B, N2), 1)
               == k2i).astype(jnp.float32)
        ure = jax.lax.dot(roh, C1, precision=_HI)
        uim = jax.lax.dot(roh, S1, precision=_HI)
        tt = jax.lax.dot(roh, CtSt, precision=_HI)
        twc, tws = tt[:, :N2], tt[:, N2:]
        cc = jax.lax.dot(coh, C2S2, precision=_HI)
        c2v, s2v = cc[:, :N2], cc[:, N2:]
        wre = twc * c2v - tws * s2v
        wim = twc * s2v + tws * c2v
        scale = (eps * (1.0 / N)).reshape(B, 1)
        a2 = a.reshape(B, 1)
        b2 = bb.reshape(B, 1)
        cure = scale * (a2 * ure - b2 * uim)
        cuim = scale * (a2 * uim + b2 * ure)
        season = (season + cure[:, :, None] * wre[:, None, :]
                  - cuim[:, :, None] * wim[:, None, :])
        mag = jnp.where(sel > 0, -1.0, mag)

    season_ref[...] = season
    trend_ref[...] = X - season


def _run(x3, interpret=False):
    nseq = x3.shape[0]
    grid = (nseq // B,)
    tabs = [jnp.asarray(t) for t in _TABLES]
    tab_specs = [pl.BlockSpec(t.shape, lambda i: (0,) * t.ndim)
                 for t in tabs]
    season3, trend3 = pl.pallas_call(
        _dft_decomp_kernel,
        grid=grid,
        in_specs=[pl.BlockSpec((B, N1, N2), lambda i: (i, 0, 0))] + tab_specs,
        out_specs=[pl.BlockSpec((B, N1, N2), lambda i: (i, 0, 0)),
                   pl.BlockSpec((B, N1, N2), lambda i: (i, 0, 0))],
        out_shape=[jax.ShapeDtypeStruct((nseq, N1, N2), jnp.float32),
                   jax.ShapeDtypeStruct((nseq, N1, N2), jnp.float32)],
        interpret=interpret,
    )(x3, *tabs)
    return season3, trend3


def kernel(x):
    bsz, ch, n = x.shape
    x3 = x.reshape(bsz * ch, N1, N2)
    season3, trend3 = _run(x3)
    return (season3.reshape(bsz, ch, n), trend3.reshape(bsz, ch, n))


# speedup vs baseline: 2.0744x; 1.0124x over previous
import numpy as np
import jax
import jax.numpy as jnp
from jax.experimental import pallas as pl
from jax.experimental.pallas import tpu as pltpu

N = 8192
N1 = 64
N2 = 128
NSEQ = 64 * 32
TOPK = 5
B = 32

_HI = jax.lax.Precision.HIGHEST


def _make_tables():
    k1 = np.arange(N1)
    n1 = np.arange(N1)
    C1 = np.cos(2 * np.pi * np.outer(k1, n1) / N1).astype(np.float32)
    S1 = np.sin(2 * np.pi * np.outer(k1, n1) / N1).astype(np.float32)
    n2 = np.arange(N2)
    Ct = np.cos(2 * np.pi * np.outer(k1, n2) / N).astype(np.float32)
    St = np.sin(2 * np.pi * np.outer(k1, n2) / N).astype(np.float32)
    k2 = np.arange(N2)
    C2 = np.cos(2 * np.pi * np.outer(n2, k2) / N2).astype(np.float32)
    S2 = np.sin(2 * np.pi * np.outer(n2, k2) / N2).astype(np.float32)
    k2h = np.arange(64)
    fgrid = (k1[:, None] + N1 * k2h[None, :]).astype(np.float32)
    valid = (fgrid >= 1).astype(np.float32)
    W2h = np.concatenate([C2[:, :64], S2[:, :64]], axis=1)
    alt = ((-1.0) ** n2).astype(np.float32)[None, :]
    BD1C = np.kron(np.eye(4, dtype=np.float32), C1)
    BD1S = np.kron(np.eye(4, dtype=np.float32), S1)
    C2S2 = np.concatenate([C2, S2], axis=1)
    C1S1 = np.concatenate([C1, S1], axis=1)
    CtSt = np.concatenate([Ct, St], axis=1)
    return C1, S1, Ct, St, C2, S2, fgrid, valid, BD1C, BD1S, C2S2, C1S1, CtSt, W2h, alt


_TABLES = _make_tables()


def _dft_decomp_kernel(x_ref, c1_ref, s1_ref, ct_ref, st_ref, c2_ref,
                       s2_ref, fg_ref, valid_ref, bd1c_ref, bd1s_ref,
                       c2s2_ref, c1s1_ref, ctst_ref, w2h_ref, alt_ref,
                       season_ref, trend_ref):
    X = x_ref[...]
    C1 = c1_ref[...]
    S1 = s1_ref[...]
    Ct = ct_ref[...]
    St = st_ref[...]
    C2 = c2_ref[...]
    S2 = s2_ref[...]
    fg = fg_ref[...]
    valid = valid_ref[...]
    BD1C = bd1c_ref[...]
    BD1S = bd1s_ref[...]
    C2S2 = c2s2_ref[...]
    C1S1 = c1s1_ref[...]
    CtSt = ctst_ref[...]
    W2h = w2h_ref[...]
    alt = alt_ref[...]

    yre_l = []
    yim_l = []
    for b in range(B):
        xb = X[b]
        yre_l.append(jax.lax.dot(C1, xb, precision=_HI)[None])
        yim_l.append(-jax.lax.dot(S1, xb, precision=_HI)[None])
    Yre = jnp.concatenate(yre_l, axis=0)
    Yim = jnp.concatenate(yim_l, axis=0)

    Ypre = Yre * Ct[None] + Yim * St[None]
    Ypim = Yim * Ct[None] - Yre * St[None]

    Ypre2 = Ypre.reshape(B * N1, N2)
    Ypim2 = Ypim.reshape(B * N1, N2)
    Pcs = jax.lax.dot(Ypre2, W2h, precision=_HI)
    Qcs = jax.lax.dot(Ypim2, W2h, precision=_HI)
    Zre2 = Pcs[:, :64] + Qcs[:, 64:]
    Zim2 = Qcs[:, :64] - Pcs[:, 64:]
    Zre = Zre2.reshape(B, N1, 64)
    Zim = Zim2.reshape(B, N1, 64)
    nyre = jnp.sum(Ypre[:, 0, :] * alt, axis=1, keepdims=True)
    nyim = jnp.sum(Ypim[:, 0, :] * alt, axis=1, keepdims=True)
    nymag = (nyre * nyre + nyim * nyim).reshape(B, 1, 1)

    mag = jnp.where(valid[None] > 0, Zre * Zre + Zim * Zim, -1.0)

    season = jnp.zeros((B, N1, N2), jnp.float32)
    takenny = jnp.zeros((B, 1, 1), jnp.bool_)
    for _ in range(TOPK):
        mm = jnp.max(mag, axis=(1, 2), keepdims=True)
        nyeff = jnp.where(takenny, -1.0, nymag)
        isny = nyeff > mm
        takenny = takenny | isny
        m = jnp.where(isny, nyeff, mm)
        sel = (mag == m).astype(jnp.float32)
        isnyf = isny.astype(jnp.float32)
        a = (jnp.sum(sel * Zre, axis=(1, 2), keepdims=True)
             + isnyf * nyre[:, :, None])
        bb = (jnp.sum(sel * Zim, axis=(1, 2), keepdims=True)
              + isnyf * nyim[:, :, None])
        fsel = (jnp.sum(sel * fg[None], axis=(1, 2), keepdims=True)
                + isnyf * float(N // 2))
        k2f = jnp.floor(fsel * (1.0 / N1))
        k1f = fsel - N1 * k2f
        eps = jnp.where(fsel == float(N // 2), 1.0, 2.0)

        k1i = k1f.reshape(B, 1).astype(jnp.int32)
        k2i = k2f.reshape(B, 1).astype(jnp.int32)
        roh = (jax.lax.broadcasted_iota(jnp.int32, (B, N1), 1)
               == k1i).astype(jnp.float32)
        coh = (jax.lax.broadcasted_iota(jnp.int32, (B, N2), 1)
               == k2i).astype(jnp.float32)
        ure = jax.lax.dot(roh, C1, precision=_HI)
        uim = jax.lax.dot(roh, S1, precision=_HI)
        tt = jax.lax.dot(roh, CtSt, precision=_HI)
        twc, tws = tt[:, :N2], tt[:, N2:]
        cc = jax.lax.dot(coh, C2S2, precision=_HI)
        c2v, s2v = cc[:, :N2], cc[:, N2:]
        wre = twc * c2v - tws * s2v
        wim = twc * s2v + tws * c2v
        scale = (eps * (1.0 / N)).reshape(B, 1)
        a2 = a.reshape(B, 1)
        b2 = bb.reshape(B, 1)
        cure = scale * (a2 * ure - b2 * uim)
        cuim = scale * (a2 * uim + b2 * ure)
        season = (season + cure[:, :, None] * wre[:, None, :]
                  - cuim[:, :, None] * wim[:, None, :])
        mag = jnp.where(sel > 0, -1.0, mag)

    season_ref[...] = season
    trend_ref[...] = X - season


def _run(x3, interpret=False):
    nseq = x3.shape[0]
    grid = (nseq // B,)
    tabs = [jnp.asarray(t) for t in _TABLES]
    tab_specs = [pl.BlockSpec(t.shape, lambda i: (0,) * t.ndim)
                 for t in tabs]
    season3, trend3 = pl.pallas_call(
        _dft_decomp_kernel,
        grid=grid,
        in_specs=[pl.BlockSpec((B, N1, N2), lambda i: (i, 0, 0))] + tab_specs,
        out_specs=[pl.BlockSpec((B, N1, N2), lambda i: (i, 0, 0)),
                   pl.BlockSpec((B, N1, N2), lambda i: (i, 0, 0))],
        out_shape=[jax.ShapeDtypeStruct((nseq, N1, N2), jnp.float32),
                   jax.ShapeDtypeStruct((nseq, N1, N2), jnp.float32)],
        interpret=interpret,
    )(x3, *tabs)
    return season3, trend3


def kernel(x):
    bsz, ch, n = x.shape
    x3 = x.reshape(bsz * ch, N1, N2)
    season3, trend3 = _run(x3)
    return (season3.reshape(bsz, ch, n), trend3.reshape(bsz, ch, n))


# speedup vs baseline: 2.0778x; 1.0016x over previous
import numpy as np
import jax
import jax.numpy as jnp
from jax.experimental import pallas as pl
from jax.experimental.pallas import tpu as pltpu

N = 8192
N1 = 64
N2 = 128
NSEQ = 64 * 32
TOPK = 5
B = 32

_HI = jax.lax.Precision.HIGHEST


def _make_tables():
    k1 = np.arange(N1)
    n1 = np.arange(N1)
    C1 = np.cos(2 * np.pi * np.outer(k1, n1) / N1).astype(np.float32)
    S1 = np.sin(2 * np.pi * np.outer(k1, n1) / N1).astype(np.float32)
    n2 = np.arange(N2)
    Ct = np.cos(2 * np.pi * np.outer(k1, n2) / N).astype(np.float32)
    St = np.sin(2 * np.pi * np.outer(k1, n2) / N).astype(np.float32)
    k2 = np.arange(N2)
    C2 = np.cos(2 * np.pi * np.outer(n2, k2) / N2).astype(np.float32)
    S2 = np.sin(2 * np.pi * np.outer(n2, k2) / N2).astype(np.float32)
    k2h = np.arange(64)
    fgrid = (k1[:, None] + N1 * k2h[None, :]).astype(np.float32)
    valid = (fgrid >= 1).astype(np.float32)
    W2h = np.concatenate([C2[:, :64], S2[:, :64]], axis=1)
    alt = ((-1.0) ** n2).astype(np.float32)[None, :]
    C2S2 = np.concatenate([C2, S2], axis=1)
    CtSt = np.concatenate([Ct, St], axis=1)
    return C1, S1, Ct, St, fgrid, valid, C2S2, CtSt, W2h, alt


_TABLES = _make_tables()


def _dft_decomp_kernel(x_ref, c1_ref, s1_ref, ct_ref, st_ref, fg_ref,
                       valid_ref, c2s2_ref, ctst_ref, w2h_ref, alt_ref,
                       season_ref, trend_ref):
    X = x_ref[...]
    C1 = c1_ref[...]
    S1 = s1_ref[...]
    Ct = ct_ref[...]
    St = st_ref[...]
    fg = fg_ref[...]
    valid = valid_ref[...]
    C2S2 = c2s2_ref[...]
    CtSt = ctst_ref[...]
    W2h = w2h_ref[...]
    alt = alt_ref[...]

    yre_l = []
    yim_l = []
    for b in range(B):
        xb = X[b]
        yre_l.append(jax.lax.dot(C1, xb, precision=_HI)[None])
        yim_l.append(-jax.lax.dot(S1, xb, precision=_HI)[None])
    Yre = jnp.concatenate(yre_l, axis=0)
    Yim = jnp.concatenate(yim_l, axis=0)

    Ypre = Yre * Ct[None] + Yim * St[None]
    Ypim = Yim * Ct[None] - Yre * St[None]

    Ypre2 = Ypre.reshape(B * N1, N2)
    Ypim2 = Ypim.reshape(B * N1, N2)
    Pcs = jax.lax.dot(Ypre2, W2h, precision=_HI)
    Qcs = jax.lax.dot(Ypim2, W2h, precision=_HI)
    Zre2 = Pcs[:, :64] + Qcs[:, 64:]
    Zim2 = Qcs[:, :64] - Pcs[:, 64:]
    Zre = Zre2.reshape(B, N1, 64)
    Zim = Zim2.reshape(B, N1, 64)
    nyre = jnp.sum(Ypre[:, 0, :] * alt, axis=1, keepdims=True)
    nyim = jnp.sum(Ypim[:, 0, :] * alt, axis=1, keepdims=True)
    nymag = (nyre * nyre + nyim * nyim).reshape(B, 1, 1)

    mag = jnp.where(valid[None] > 0, Zre * Zre + Zim * Zim, -1.0)

    season = jnp.zeros((B, N1, N2), jnp.float32)
    takenny = jnp.zeros((B, 1, 1), jnp.bool_)
    for _ in range(TOPK):
        mm = jnp.max(mag, axis=(1, 2), keepdims=True)
        nyeff = jnp.where(takenny, -1.0, nymag)
        isny = nyeff > mm
        takenny = takenny | isny
        m = jnp.where(isny, nyeff, mm)
        sel = (mag == m).astype(jnp.float32)
        isnyf = isny.astype(jnp.float32)
        a = (jnp.sum(sel * Zre, axis=(1, 2), keepdims=True)
             + isnyf * nyre[:, :, None])
        bb = (jnp.sum(sel * Zim, axis=(1, 2), keepdims=True)
              + isnyf * nyim[:, :, None])
        fsel = (jnp.sum(sel * fg[None], axis=(1, 2), keepdims=True)
                + isnyf * float(N // 2))
        k2f = jnp.floor(fsel * (1.0 / N1))
        k1f = fsel - N1 * k2f
        eps = jnp.where(fsel == float(N // 2), 1.0, 2.0)

        k1i = k1f.reshape(B, 1).astype(jnp.int32)
        k2i = k2f.reshape(B, 1).astype(jnp.int32)
        roh = (jax.lax.broadcasted_iota(jnp.int32, (B, N1), 1)
               == k1i).astype(jnp.float32)
        coh = (jax.lax.broadcasted_iota(jnp.int32, (B, N2), 1)
               == k2i).astype(jnp.float32)
        ure = jax.lax.dot(roh, C1, precision=_HI)
        uim = jax.lax.dot(roh, S1, precision=_HI)
        tt = jax.lax.dot(roh, CtSt, precision=_HI)
        twc, tws = tt[:, :N2], tt[:, N2:]
        cc = jax.lax.dot(coh, C2S2, precision=_HI)
        c2v, s2v = cc[:, :N2], cc[:, N2:]
        wre = twc * c2v - tws * s2v
        wim = twc * s2v + tws * c2v
        scale = (eps * (1.0 / N)).reshape(B, 1)
        a2 = a.reshape(B, 1)
        b2 = bb.reshape(B, 1)
        cure = scale * (a2 * ure - b2 * uim)
        cuim = scale * (a2 * uim + b2 * ure)
        season = (season + cure[:, :, None] * wre[:, None, :]
                  - cuim[:, :, None] * wim[:, None, :])
        mag = jnp.where(sel > 0, -1.0, mag)

    season_ref[...] = season
    trend_ref[...] = X - season


def _run(x3, interpret=False):
    nseq = x3.shape[0]
    grid = (nseq // B,)
    tabs = [jnp.asarray(t) for t in _TABLES]
    tab_specs = [pl.BlockSpec(t.shape, lambda i: (0,) * t.ndim)
                 for t in tabs]
    season3, trend3 = pl.pallas_call(
        _dft_decomp_kernel,
        grid=grid,
        in_specs=[pl.BlockSpec((B, N1, N2), lambda i: (i, 0, 0))] + tab_specs,
        out_specs=[pl.BlockSpec((B, N1, N2), lambda i: (i, 0, 0)),
                   pl.BlockSpec((B, N1, N2), lambda i: (i, 0, 0))],
        out_shape=[jax.ShapeDtypeStruct((nseq, N1, N2), jnp.float32),
                   jax.ShapeDtypeStruct((nseq, N1, N2), jnp.float32)],
        interpret=interpret,
    )(x3, *tabs)
    return season3, trend3


def kernel(x):
    bsz, ch, n = x.shape
    x3 = x.reshape(bsz * ch, N1, N2)
    season3, trend3 = _run(x3)
    return (season3.reshape(bsz, ch, n), trend3.reshape(bsz, ch, n))
